# baseline (speedup 1.0000x reference)
import jax
import jax.numpy as jnp
from jax import lax
from jax.experimental import pallas as pl
from jax.experimental.pallas import tpu as pltpu
from jax.experimental.pallas import tpu_sc as plsc

ROWS, COLS = 4096, 50
EMBED_DIM = 128
B = ROWS * COLS

NUM_CORES = 2
NUM_SUBCORES = 16
NW = NUM_CORES * NUM_SUBCORES
B_PER_W = B // NW
CHUNK = 80
NCHUNK = B_PER_W // CHUNK
NBUF = 4
K = 2
NGROUP = NCHUNK // NBUF


def _embed_body(x_hbm, w_hbm, out_hbm, idx_v, *rest):
    bufs = rest[:NBUF]
    gsem = rest[NBUF : 2 * NBUF]
    ssem = rest[2 * NBUF : 3 * NBUF]

    wid = lax.axis_index("s") * NUM_CORES + lax.axis_index("c")
    base = wid * B_PER_W
    pltpu.sync_copy(x_hbm.at[pl.ds(base, B_PER_W)], idx_v)

    def gather(j, b):
        pltpu.async_copy(
            w_hbm.at[idx_v.at[pl.ds(j * CHUNK, CHUNK)]], bufs[b], gsem[b]
        )

    def wait_gather(b):
        pltpu.make_async_copy(w_hbm.at[pl.ds(0, CHUNK)], bufs[b], gsem[b]).wait()

    def store(j, b):
        pltpu.async_copy(bufs[b], out_hbm.at[pl.ds(base + j * CHUNK, CHUNK)], ssem[b])

    def wait_store(b):
        pltpu.make_async_copy(w_hbm.at[pl.ds(0, CHUNK)], bufs[b], ssem[b]).wait()

    for j in range(K):
        gather(j, j % NBUF)

    for v in range(NBUF):
        b2 = (v + K) % NBUF
        if v + K >= NBUF:
            wait_store(b2)
        gather(v + K, b2)
        wait_gather(v % NBUF)
        store(v, v % NBUF)

    def group(g, carry):
        j0 = g * NBUF
        for v in range(NBUF):
            j = j0 + v
            b = v % NBUF
            b2 = (v + K) % NBUF
            wait_store(b2)
            gather(j + K, b2)
            wait_gather(b)
            store(j, b)
        return carry

    lax.fori_loop(1, NGROUP - 1, group, 0)

    j0 = (NGROUP - 1) * NBUF
    for v in range(NBUF):
        j = j0 + v
        b = v % NBUF
        if j + K < NCHUNK:
            b2 = (v + K) % NBUF
            wait_store(b2)
            gather(j + K, b2)
        wait_gather(b)
        store(j, b)

    for b in range(NBUF):
        wait_store(b)


@jax.jit
def _embed(x_flat, W):
    mesh = plsc.VectorSubcoreMesh(core_axis_name="c", subcore_axis_name="s")
    run = pl.kernel(
        _embed_body,
        mesh=mesh,
        out_type=jax.ShapeDtypeStruct((B, EMBED_DIM), jnp.float32),
        scratch_types=[
            pltpu.VMEM((B_PER_W,), jnp.int32),
            *[pltpu.VMEM((CHUNK, EMBED_DIM), jnp.float32) for _ in range(NBUF)],
            *[pltpu.SemaphoreType.DMA for _ in range(2 * NBUF)],
        ],
    )
    return run(x_flat, W)


def kernel(x, W):
    x_flat = jnp.swapaxes(x, 0, 1).reshape(B).astype(jnp.int32)
    out = _embed(x_flat, W)
    return jnp.swapaxes(out.reshape(COLS, ROWS, EMBED_DIM), 0, 1)


# speedup vs baseline: 1.0000x; 1.0000x over previous
import jax
import jax.numpy as jnp
from jax import lax
from jax.experimental import pallas as pl
from jax.experimental.pallas import tpu as pltpu
from jax.experimental.pallas import tpu_sc as plsc

ROWS, COLS = 4096, 50
EMBED_DIM = 128
B = ROWS * COLS

NUM_CORES = 2
NUM_SUBCORES = 16
NW = NUM_CORES * NUM_SUBCORES
B_PER_W = B // NW
CHUNK = 200
NCHUNK = B_PER_W // CHUNK
NBUF = 4
K = 2
NGROUP = NCHUNK // NBUF


def _embed_body(x_hbm, w_hbm, out_hbm, idx_v, *rest):
    bufs = rest[:NBUF]
    gsem = rest[NBUF : 2 * NBUF]
    ssem = rest[2 * NBUF : 3 * NBUF]

    wid = lax.axis_index("s") * NUM_CORES + lax.axis_index("c")
    base = wid * B_PER_W
    pltpu.sync_copy(x_hbm.at[pl.ds(base, B_PER_W)], idx_v)

    def gather(j, b):
        pltpu.async_copy(
            w_hbm.at[idx_v.at[pl.ds(j * CHUNK, CHUNK)]], bufs[b], gsem[b]
        )

    def wait_gather(b):
        pltpu.make_async_copy(w_hbm.at[pl.ds(0, CHUNK)], bufs[b], gsem[b]).wait()

    def store(j, b):
        pltpu.async_copy(bufs[b], out_hbm.at[pl.ds(base + j * CHUNK, CHUNK)], ssem[b])

    def wait_store(b):
        pltpu.make_async_copy(w_hbm.at[pl.ds(0, CHUNK)], bufs[b], ssem[b]).wait()

    for j in range(K):
        gather(j, j % NBUF)

    for v in range(NBUF):
        b2 = (v + K) % NBUF
        if v + K >= NBUF:
            wait_store(b2)
        gather(v + K, b2)
        wait_gather(v % NBUF)
        store(v, v % NBUF)

    def group(g, carry):
        j0 = g * NBUF
        for v in range(NBUF):
            j = j0 + v
            b = v % NBUF
            b2 = (v + K) % NBUF
            wait_store(b2)
            gather(j + K, b2)
            wait_gather(b)
            store(j, b)
        return carry

    lax.fori_loop(1, NGROUP - 1, group, 0)

    j0 = (NGROUP - 1) * NBUF
    for v in range(NBUF):
        j = j0 + v
        b = v % NBUF
        if j + K < NCHUNK:
            b2 = (v + K) % NBUF
            wait_store(b2)
            gather(j + K, b2)
        wait_gather(b)
        store(j, b)

    for b in range(NBUF):
        wait_store(b)


@jax.jit
def _embed(x_flat, W):
    mesh = plsc.VectorSubcoreMesh(core_axis_name="c", subcore_axis_name="s")
    run = pl.kernel(
        _embed_body,
        mesh=mesh,
        out_type=jax.ShapeDtypeStruct((B, EMBED_DIM), jnp.float32),
        scratch_types=[
            pltpu.VMEM((B_PER_W,), jnp.int32),
            *[pltpu.VMEM((CHUNK, EMBED_DIM), jnp.float32) for _ in range(NBUF)],
            *[pltpu.SemaphoreType.DMA for _ in range(2 * NBUF)],
        ],
    )
    return run(x_flat, W)


def kernel(x, W):
    x_flat = jnp.swapaxes(x, 0, 1).reshape(B).astype(jnp.int32)
    out = _embed(x_flat, W)
    return jnp.swapaxes(out.reshape(COLS, ROWS, EMBED_DIM), 0, 1)
